# Initial kernel scaffold; baseline (speedup 1.0000x reference)
import jax
import jax.numpy as jnp
from jax.experimental import pallas as pl


def kernel(x, edge_index, edge_type, weights, roots, biases):
    raise NotImplementedError("write your pallas kernel here")


# speedup vs baseline: 17.9126x; 17.9126x over previous
import dataclasses
import functools

import jax
import jax.numpy as jnp
from jax import lax
from jax.experimental import pallas as pl
from jax.experimental.pallas import tpu as pltpu
from jax.experimental.pallas import tpu_sc as plsc

N = 10000
E = 320000
D = 128
R = 3
L = 6

NC = 2
NS = 16
NW = NC * NS
EPT = E // NW
CHUNK = 200
NPT = 624
NRPAD = 30720
SPT = NRPAD // NS

_mesh = plsc.VectorSubcoreMesh(core_axis_name="c", subcore_axis_name="s")

_sc_params = pltpu.CompilerParams()
if "needs_layout_passes" in pltpu.CompilerParams.__dataclass_fields__:
    _sc_params = dataclasses.replace(_sc_params, needs_layout_passes=False)


def _zero16():
    return jnp.zeros((16,), jnp.float32)


@functools.partial(
    pl.kernel,
    out_type=jax.ShapeDtypeStruct((NC, NRPAD), jnp.float32),
    mesh=_mesh,
    scratch_types=[
        pltpu.VMEM((CHUNK,), jnp.int32),
        pltpu.VMEM((CHUNK,), jnp.float32),
        pltpu.VMEM((SPT,), jnp.float32),
        pltpu.VMEM_SHARED((NRPAD,), jnp.float32),
    ],
)
def _k1_counts(comb_hbm, cnt_hbm, cidx_v, ones_v, zer_v, cnt_sh):
    c = lax.axis_index("c")
    s = lax.axis_index("s")
    wid = c * NS + s

    @pl.loop(0, CHUNK, step=16)
    def _(i):
        ones_v[pl.ds(i, 16)] = jnp.ones((16,), jnp.float32)

    @pl.loop(0, SPT, step=16)
    def _(i):
        zer_v[pl.ds(i, 16)] = _zero16()

    pltpu.sync_copy(zer_v, cnt_sh.at[pl.ds(s * SPT, SPT)])
    plsc.subcore_barrier()

    @pl.loop(0, EPT, step=CHUNK)
    def _(e0):
        pltpu.sync_copy(comb_hbm.at[pl.ds(wid * EPT + e0, CHUNK)], cidx_v)
        pltpu.sync_copy(ones_v, cnt_sh.at[cidx_v], add=True)

    plsc.subcore_barrier()
    pltpu.sync_copy(cnt_sh.at[pl.ds(s * SPT, SPT)],
                    cnt_hbm.at[c, pl.ds(s * SPT, SPT)])


@functools.partial(
    pl.kernel,
    out_type=jax.ShapeDtypeStruct((E,), jnp.float32),
    mesh=_mesh,
    scratch_types=[
        pltpu.VMEM((NRPAD,), jnp.float32),
        pltpu.VMEM((NRPAD,), jnp.float32),
        pltpu.VMEM((EPT,), jnp.int32),
        pltpu.VMEM((EPT,), jnp.float32),
    ],
    compiler_params=_sc_params,
)
def _k2_weights(cnt_hbm, comb_hbm, w_hbm, c0_v, c1_v, comb_v, w_v):
    c = lax.axis_index("c")
    s = lax.axis_index("s")
    wid = c * NS + s
    pltpu.sync_copy(cnt_hbm.at[0], c0_v)
    pltpu.sync_copy(cnt_hbm.at[1], c1_v)
    pltpu.sync_copy(comb_hbm.at[pl.ds(wid * EPT, EPT)], comb_v)

    @pl.loop(0, EPT, step=16)
    def _(i):
        cv = comb_v[pl.ds(i, 16)]
        g0 = plsc.load_gather(c0_v, [cv])
        g1 = plsc.load_gather(c1_v, [cv])
        w_v[pl.ds(i, 16)] = 1.0 / jnp.maximum(g0 + g1, 1.0)

    pltpu.sync_copy(w_v, w_hbm.at[pl.ds(wid * EPT, EPT)])


@functools.partial(
    pl.kernel,
    out_type=jax.ShapeDtypeStruct((NC, N, D), jnp.float32),
    mesh=_mesh,
    scratch_types=[
        pltpu.VMEM((CHUNK,), jnp.int32),
        pltpu.VMEM((CHUNK,), jnp.int32),
        pltpu.VMEM((CHUNK,), jnp.float32),
        pltpu.VMEM((CHUNK, D), jnp.float32),
        pltpu.VMEM_SHARED((N, D), jnp.float32),
    ],
    compiler_params=_sc_params,
)
def _k3_aggregate(hrw_hbm, fidx_hbm, dst_hbm, w_hbm, bsum_hbm,
                  gidx_v, sidx_v, w_v, rows_v, bkt_sh):
    c = lax.axis_index("c")
    s = lax.axis_index("s")
    wid = c * NS + s
    base = wid * EPT

    @pl.loop(0, CHUNK)
    def _(i):
        for j in range(D // 16):
            rows_v[i, pl.ds(j * 16, 16)] = _zero16()

    def _zero_rows(start, cnt):
        off = 0
        while cnt - off >= CHUNK:
            pltpu.sync_copy(rows_v, bkt_sh.at[pl.ds(start + off, CHUNK)])
            off += CHUNK
        if cnt > off:
            pltpu.sync_copy(rows_v.at[pl.ds(0, cnt - off)],
                            bkt_sh.at[pl.ds(start + off, cnt - off)])

    _zero_rows(s * NPT, NPT)

    @pl.when(s == NS - 1)
    def _():
        _zero_rows(NS * NPT, N - NS * NPT)

    plsc.subcore_barrier()

    @pl.loop(0, EPT, step=CHUNK)
    def _(e0):
        pltpu.sync_copy(fidx_hbm.at[pl.ds(base + e0, CHUNK)], gidx_v)
        pltpu.sync_copy(dst_hbm.at[pl.ds(base + e0, CHUNK)], sidx_v)
        pltpu.sync_copy(w_hbm.at[pl.ds(base + e0, CHUNK)], w_v)
        pltpu.sync_copy(hrw_hbm.at[gidx_v], rows_v)

        @pl.loop(0, CHUNK)
        def _(i):
            ww = plsc.load_gather(w_v, [jnp.full((16,), i, jnp.int32)])
            for j in range(D // 16):
                sl = pl.ds(j * 16, 16)
                rows_v[i, sl] = rows_v[i, sl] * ww

        pltpu.sync_copy(rows_v, bkt_sh.at[sidx_v], add=True)

    plsc.subcore_barrier()
    pltpu.sync_copy(bkt_sh.at[pl.ds(s * NPT, NPT)],
                    bsum_hbm.at[c, pl.ds(s * NPT, NPT)])

    @pl.when(s == NS - 1)
    def _():
        pltpu.sync_copy(bkt_sh.at[pl.ds(NS * NPT, N - NS * NPT)],
                        bsum_hbm.at[c, pl.ds(NS * NPT, N - NS * NPT)])


_NB = 10
_BN = N // _NB


def _tca_body(x_ref, w_ref, o_ref):
    o_ref[0] = jnp.dot(x_ref[...], w_ref[0],
                       preferred_element_type=jnp.float32)


def _tca(h, w):
    return pl.pallas_call(
        _tca_body,
        grid=(R, _NB),
        in_specs=[
            pl.BlockSpec((_BN, D), lambda r, i: (i, 0)),
            pl.BlockSpec((1, D, D), lambda r, i: (r, 0, 0)),
        ],
        out_specs=pl.BlockSpec((1, _BN, D), lambda r, i: (r, i, 0)),
        out_shape=jax.ShapeDtypeStruct((R, N, D), jnp.float32),
    )(h, w)


def _tcb_body(bsum_ref, h_ref, root_ref, b_ref, wn_ref, h_out, hrw_out):
    hn = bsum_ref[0] + bsum_ref[1]
    hn = hn + jnp.dot(h_ref[...], root_ref[...],
                      preferred_element_type=jnp.float32)
    hn = jnp.maximum(hn + b_ref[...], 0.0)
    h_out[...] = hn
    if hrw_out is not None:
        for r in range(R):
            hrw_out[r] = jnp.dot(hn, wn_ref[r],
                                 preferred_element_type=jnp.float32)


def _tcb(bsum, h, root, b, wnext):
    b2 = b.reshape(1, D)
    in_specs = [
        pl.BlockSpec((NC, _BN, D), lambda i: (0, i, 0)),
        pl.BlockSpec((_BN, D), lambda i: (i, 0)),
        pl.BlockSpec((D, D), lambda i: (0, 0)),
        pl.BlockSpec((1, D), lambda i: (0, 0)),
    ]
    if wnext is None:
        body = lambda bs, hh, rt, bb, ho: _tcb_body(bs, hh, rt, bb, None,
                                                    ho, None)
        return pl.pallas_call(
            body,
            grid=(_NB,),
            in_specs=in_specs,
            out_specs=pl.BlockSpec((_BN, D), lambda i: (i, 0)),
            out_shape=jax.ShapeDtypeStruct((N, D), jnp.float32),
        )(bsum, h, root, b2)
    return pl.pallas_call(
        _tcb_body,
        grid=(_NB,),
        in_specs=in_specs + [pl.BlockSpec((R, D, D), lambda i: (0, 0, 0))],
        out_specs=[
            pl.BlockSpec((_BN, D), lambda i: (i, 0)),
            pl.BlockSpec((R, _BN, D), lambda i: (0, i, 0)),
        ],
        out_shape=[
            jax.ShapeDtypeStruct((N, D), jnp.float32),
            jax.ShapeDtypeStruct((R, N, D), jnp.float32),
        ],
    )(bsum, h, root, b2, wnext)


def kernel(x, edge_index, edge_type, weights, roots, biases):
    src = edge_index[0].astype(jnp.int32)
    dst = edge_index[1].astype(jnp.int32)
    et = edge_type.astype(jnp.int32)
    comb = dst * R + et
    fidx = et * N + src

    cnt = _k1_counts(comb)
    w = _k2_weights(cnt, comb)

    h = x
    hrw = _tca(h, weights[0]).reshape(R * N, D)
    for l in range(L):
        bsum = _k3_aggregate(hrw, fidx, dst, w)
        if l < L - 1:
            h, hrw3 = _tcb(bsum, h, roots[l], biases[l], weights[l + 1])
            hrw = hrw3.reshape(R * N, D)
        else:
            h = _tcb(bsum, h, roots[l], biases[l], None)
    return h


# speedup vs baseline: 30.6882x; 1.7132x over previous
import dataclasses
import functools

import jax
import jax.numpy as jnp
from jax import lax
from jax.experimental import pallas as pl
from jax.experimental.pallas import tpu as pltpu
from jax.experimental.pallas import tpu_sc as plsc

N = 10000
E = 320000
D = 128
R = 3
L = 6

NC = 2
NS = 16
NW = NC * NS
EPT = E // NW
CHUNK = 200
KCH = 128
EPTP = 10240
NCH = EPTP // KCH
EP = NW * EPTP
NPT = 624
NRPAD = 30720
SPT = NRPAD // NS

_mesh = plsc.VectorSubcoreMesh(core_axis_name="c", subcore_axis_name="s")

_sc_params = pltpu.CompilerParams()
if "needs_layout_passes" in pltpu.CompilerParams.__dataclass_fields__:
    _sc_params = dataclasses.replace(_sc_params, needs_layout_passes=False)


def _zero16():
    return jnp.zeros((16,), jnp.float32)


@functools.partial(
    pl.kernel,
    out_type=jax.ShapeDtypeStruct((NC, NRPAD), jnp.float32),
    mesh=_mesh,
    scratch_types=[
        pltpu.VMEM((CHUNK,), jnp.int32),
        pltpu.VMEM((CHUNK,), jnp.float32),
        pltpu.VMEM((SPT,), jnp.float32),
        pltpu.VMEM_SHARED((NRPAD,), jnp.float32),
    ],
)
def _k1_counts(comb_hbm, cnt_hbm, cidx_v, ones_v, zer_v, cnt_sh):
    c = lax.axis_index("c")
    s = lax.axis_index("s")
    wid = c * NS + s

    @pl.loop(0, CHUNK, step=16)
    def _(i):
        ones_v[pl.ds(i, 16)] = jnp.ones((16,), jnp.float32)

    @pl.loop(0, SPT, step=16)
    def _(i):
        zer_v[pl.ds(i, 16)] = _zero16()

    pltpu.sync_copy(zer_v, cnt_sh.at[pl.ds(s * SPT, SPT)])
    plsc.subcore_barrier()

    @pl.loop(0, EPT, step=CHUNK)
    def _(e0):
        pltpu.sync_copy(comb_hbm.at[pl.ds(wid * EPT + e0, CHUNK)], cidx_v)
        pltpu.sync_copy(ones_v, cnt_sh.at[cidx_v], add=True)

    plsc.subcore_barrier()
    pltpu.sync_copy(cnt_sh.at[pl.ds(s * SPT, SPT)],
                    cnt_hbm.at[c, pl.ds(s * SPT, SPT)])


@functools.partial(
    pl.kernel,
    out_type=jax.ShapeDtypeStruct((E,), jnp.float32),
    mesh=_mesh,
    scratch_types=[
        pltpu.VMEM((NRPAD,), jnp.float32),
        pltpu.VMEM((NRPAD,), jnp.float32),
        pltpu.VMEM((EPT,), jnp.int32),
        pltpu.VMEM((EPT,), jnp.float32),
    ],
    compiler_params=_sc_params,
)
def _k2_weights(cnt_hbm, comb_hbm, w_hbm, c0_v, c1_v, comb_v, w_v):
    c = lax.axis_index("c")
    s = lax.axis_index("s")
    wid = c * NS + s
    pltpu.sync_copy(cnt_hbm.at[0], c0_v)
    pltpu.sync_copy(cnt_hbm.at[1], c1_v)
    pltpu.sync_copy(comb_hbm.at[pl.ds(wid * EPT, EPT)], comb_v)

    @pl.loop(0, EPT, step=16)
    def _(i):
        cv = comb_v[pl.ds(i, 16)]
        g0 = plsc.load_gather(c0_v, [cv])
        g1 = plsc.load_gather(c1_v, [cv])
        w_v[pl.ds(i, 16)] = 1.0 / jnp.maximum(g0 + g1, 1.0)

    pltpu.sync_copy(w_v, w_hbm.at[pl.ds(wid * EPT, EPT)])


@functools.partial(
    pl.kernel,
    out_type=jax.ShapeDtypeStruct((NC, N, D), jnp.float32),
    mesh=_mesh,
    scratch_types=[
        [pltpu.VMEM((KCH,), jnp.int32) for _ in range(3)],
        [pltpu.VMEM((KCH,), jnp.int32) for _ in range(3)],
        [pltpu.VMEM((KCH,), jnp.float32) for _ in range(3)],
        [pltpu.VMEM((KCH, D), jnp.float32) for _ in range(3)],
        pltpu.VMEM_SHARED((N, D), jnp.float32),
        [pltpu.SemaphoreType.DMA for _ in range(3)],
        [pltpu.SemaphoreType.DMA for _ in range(3)],
    ],
    compiler_params=_sc_params,
)
def _k3_aggregate(hrw_hbm, fidx_hbm, dst_hbm, w_hbm, bsum_hbm,
                  gidx, sidx, wv, rows, bkt_sh, rsem, isem):
    c = lax.axis_index("c")
    s = lax.axis_index("s")
    wid = c * NS + s
    base = wid * EPTP

    @pl.loop(0, KCH)
    def _(i):
        for j in range(D // 16):
            rows[0][i, pl.ds(j * 16, 16)] = _zero16()

    def _zero_rows(start, cnt):
        off = 0
        while cnt - off >= KCH:
            pltpu.sync_copy(rows[0], bkt_sh.at[pl.ds(start + off, KCH)])
            off += KCH
        if cnt > off:
            pltpu.sync_copy(rows[0].at[pl.ds(0, cnt - off)],
                            bkt_sh.at[pl.ds(start + off, cnt - off)])

    _zero_rows(s * NPT, NPT)

    @pl.when(s == NS - 1)
    def _():
        _zero_rows(NS * NPT, N - NS * NPT)

    plsc.subcore_barrier()

    def _idx_start(k, x):
        off = base + k * KCH
        pltpu.async_copy(fidx_hbm.at[pl.ds(off, KCH)], gidx[x], isem[x])
        pltpu.async_copy(dst_hbm.at[pl.ds(off, KCH)], sidx[x], isem[x])
        pltpu.async_copy(w_hbm.at[pl.ds(off, KCH)], wv[x], isem[x])

    def _idx_wait(k, x):
        off = base + k * KCH
        pltpu.make_async_copy(fidx_hbm.at[pl.ds(off, KCH)], gidx[x],
                              isem[x]).wait()
        pltpu.make_async_copy(dst_hbm.at[pl.ds(off, KCH)], sidx[x],
                              isem[x]).wait()
        pltpu.make_async_copy(w_hbm.at[pl.ds(off, KCH)], wv[x],
                              isem[x]).wait()

    def _gather_start(x):
        pltpu.async_copy(hrw_hbm.at[gidx[x]], rows[x], rsem[x])

    def _visit(k, x, y, prefetch=True):
        pltpu.make_async_copy(hrw_hbm.at[gidx[x]], rows[x], rsem[x]).wait()
        buf = rows[x]

        @pl.loop(0, KCH)
        def _(i):
            ww = plsc.load_gather(wv[x], [jnp.full((16,), i, jnp.int32)])
            for jj in range(D // 16):
                sl = pl.ds(jj * 16, 16)
                buf[i, sl] = buf[i, sl] * ww

        pltpu.sync_copy(buf, bkt_sh.at[sidx[x]], add=True)

        if prefetch:
            @pl.when(k + 3 < NCH)
            def _():
                _idx_start(k + 3, x)

            _idx_wait(k + 2, y)
            _gather_start(y)

    _idx_start(0, 0)
    _idx_start(1, 1)
    _idx_start(2, 2)
    _idx_wait(0, 0)
    _gather_start(0)
    _idx_wait(1, 1)
    _gather_start(1)

    _NTRIPS = (NCH // 3) * 3

    @pl.loop(0, _NTRIPS, step=3)
    def _(k):
        _visit(k, 0, 2)
        _visit(k + 1, 1, 0)
        _visit(k + 2, 2, 1)

    _visit(_NTRIPS, 0, 2, prefetch=False)
    _visit(_NTRIPS + 1, 1, 0, prefetch=False)

    plsc.subcore_barrier()
    pltpu.sync_copy(bkt_sh.at[pl.ds(s * NPT, NPT)],
                    bsum_hbm.at[c, pl.ds(s * NPT, NPT)])

    @pl.when(s == NS - 1)
    def _():
        pltpu.sync_copy(bkt_sh.at[pl.ds(NS * NPT, N - NS * NPT)],
                        bsum_hbm.at[c, pl.ds(NS * NPT, N - NS * NPT)])


_NB = 10
_BN = N // _NB


def _tca_body(x_ref, w_ref, o_ref):
    o_ref[0] = jnp.dot(x_ref[...], w_ref[0],
                       preferred_element_type=jnp.float32)


def _tca(h, w):
    return pl.pallas_call(
        _tca_body,
        grid=(R, _NB),
        in_specs=[
            pl.BlockSpec((_BN, D), lambda r, i: (i, 0)),
            pl.BlockSpec((1, D, D), lambda r, i: (r, 0, 0)),
        ],
        out_specs=pl.BlockSpec((1, _BN, D), lambda r, i: (r, i, 0)),
        out_shape=jax.ShapeDtypeStruct((R, N, D), jnp.float32),
    )(h, w)


def _tcb_body(bsum_ref, h_ref, root_ref, b_ref, wn_ref, h_out, hrw_out):
    hn = bsum_ref[0] + bsum_ref[1]
    hn = hn + jnp.dot(h_ref[...], root_ref[...],
                      preferred_element_type=jnp.float32)
    hn = jnp.maximum(hn + b_ref[...], 0.0)
    h_out[...] = hn
    if hrw_out is not None:
        for r in range(R):
            hrw_out[r] = jnp.dot(hn, wn_ref[r],
                                 preferred_element_type=jnp.float32)


def _tcb(bsum, h, root, b, wnext):
    b2 = b.reshape(1, D)
    in_specs = [
        pl.BlockSpec((NC, _BN, D), lambda i: (0, i, 0)),
        pl.BlockSpec((_BN, D), lambda i: (i, 0)),
        pl.BlockSpec((D, D), lambda i: (0, 0)),
        pl.BlockSpec((1, D), lambda i: (0, 0)),
    ]
    if wnext is None:
        body = lambda bs, hh, rt, bb, ho: _tcb_body(bs, hh, rt, bb, None,
                                                    ho, None)
        return pl.pallas_call(
            body,
            grid=(_NB,),
            in_specs=in_specs,
            out_specs=pl.BlockSpec((_BN, D), lambda i: (i, 0)),
            out_shape=jax.ShapeDtypeStruct((N, D), jnp.float32),
        )(bsum, h, root, b2)
    return pl.pallas_call(
        _tcb_body,
        grid=(_NB,),
        in_specs=in_specs + [pl.BlockSpec((R, D, D), lambda i: (0, 0, 0))],
        out_specs=[
            pl.BlockSpec((_BN, D), lambda i: (i, 0)),
            pl.BlockSpec((R, _BN, D), lambda i: (0, i, 0)),
        ],
        out_shape=[
            jax.ShapeDtypeStruct((N, D), jnp.float32),
            jax.ShapeDtypeStruct((R, N, D), jnp.float32),
        ],
    )(bsum, h, root, b2, wnext)


def kernel(x, edge_index, edge_type, weights, roots, biases):
    src = edge_index[0].astype(jnp.int32)
    dst = edge_index[1].astype(jnp.int32)
    et = edge_type.astype(jnp.int32)
    comb = dst * R + et
    fidx = et * N + src

    cnt = _k1_counts(comb)
    w = _k2_weights(cnt, comb)

    npad = EPTP - EPT
    pad_g = (jnp.arange(npad, dtype=jnp.int32)[None, :] * 131
             + jnp.arange(NW, dtype=jnp.int32)[:, None] * 17) % (R * N)
    pad_s = (jnp.arange(npad, dtype=jnp.int32)[None, :] * 41
             + jnp.arange(NW, dtype=jnp.int32)[:, None] * 13) % N
    pad_w = jnp.zeros((NW, npad), jnp.float32)
    fidx_p = jnp.concatenate([fidx.reshape(NW, EPT), pad_g], axis=1).reshape(EP)
    dst_p = jnp.concatenate([dst.reshape(NW, EPT), pad_s], axis=1).reshape(EP)
    w_p = jnp.concatenate([w.reshape(NW, EPT), pad_w], axis=1).reshape(EP)

    h = x
    hrw = _tca(h, weights[0]).reshape(R * N, D)
    for l in range(L):
        bsum = _k3_aggregate(hrw, fidx_p, dst_p, w_p)
        if l < L - 1:
            h, hrw3 = _tcb(bsum, h, roots[l], biases[l], weights[l + 1])
            hrw = hrw3.reshape(R * N, D)
        else:
            h = _tcb(bsum, h, roots[l], biases[l], None)
    return h


# speedup vs baseline: 37.7960x; 1.2316x over previous
import dataclasses
import functools

import jax
import jax.numpy as jnp
from jax import lax
from jax.experimental import pallas as pl
from jax.experimental.pallas import tpu as pltpu
from jax.experimental.pallas import tpu_sc as plsc

N = 10000
E = 320000
D = 128
R = 3
L = 6

NC = 2
NS = 16
NW = NC * NS
EPT = E // NW
CHUNK = 200
KCH = 128
EPTP = 10240
NCH = EPTP // KCH
EP = NW * EPTP
NPT = 624
NRPAD = 30720
SPT = NRPAD // NS

_mesh = plsc.VectorSubcoreMesh(core_axis_name="c", subcore_axis_name="s")

_sc_params = pltpu.CompilerParams()
if "needs_layout_passes" in pltpu.CompilerParams.__dataclass_fields__:
    _sc_params = dataclasses.replace(_sc_params, needs_layout_passes=False)


def _zero16():
    return jnp.zeros((16,), jnp.float32)


@functools.partial(
    pl.kernel,
    out_type=jax.ShapeDtypeStruct((NC, NRPAD), jnp.float32),
    mesh=_mesh,
    scratch_types=[
        pltpu.VMEM((CHUNK,), jnp.int32),
        pltpu.VMEM((CHUNK,), jnp.float32),
        pltpu.VMEM((SPT,), jnp.float32),
        pltpu.VMEM_SHARED((NRPAD,), jnp.float32),
    ],
)
def _k1_counts(comb_hbm, cnt_hbm, cidx_v, ones_v, zer_v, cnt_sh):
    c = lax.axis_index("c")
    s = lax.axis_index("s")
    wid = c * NS + s

    @pl.loop(0, CHUNK, step=16)
    def _(i):
        ones_v[pl.ds(i, 16)] = jnp.ones((16,), jnp.float32)

    @pl.loop(0, SPT, step=16)
    def _(i):
        zer_v[pl.ds(i, 16)] = _zero16()

    pltpu.sync_copy(zer_v, cnt_sh.at[pl.ds(s * SPT, SPT)])
    plsc.subcore_barrier()

    @pl.loop(0, EPT, step=CHUNK)
    def _(e0):
        pltpu.sync_copy(comb_hbm.at[pl.ds(wid * EPT + e0, CHUNK)], cidx_v)
        pltpu.sync_copy(ones_v, cnt_sh.at[cidx_v], add=True)

    plsc.subcore_barrier()
    pltpu.sync_copy(cnt_sh.at[pl.ds(s * SPT, SPT)],
                    cnt_hbm.at[c, pl.ds(s * SPT, SPT)])


@functools.partial(
    pl.kernel,
    out_type=jax.ShapeDtypeStruct((E,), jnp.float32),
    mesh=_mesh,
    scratch_types=[
        pltpu.VMEM((NRPAD,), jnp.float32),
        pltpu.VMEM((NRPAD,), jnp.float32),
        pltpu.VMEM((EPT,), jnp.int32),
        pltpu.VMEM((EPT,), jnp.float32),
    ],
    compiler_params=_sc_params,
)
def _k2_weights(cnt_hbm, comb_hbm, w_hbm, c0_v, c1_v, comb_v, w_v):
    c = lax.axis_index("c")
    s = lax.axis_index("s")
    wid = c * NS + s
    pltpu.sync_copy(cnt_hbm.at[0], c0_v)
    pltpu.sync_copy(cnt_hbm.at[1], c1_v)
    pltpu.sync_copy(comb_hbm.at[pl.ds(wid * EPT, EPT)], comb_v)

    @pl.loop(0, EPT, step=16)
    def _(i):
        cv = comb_v[pl.ds(i, 16)]
        g0 = plsc.load_gather(c0_v, [cv])
        g1 = plsc.load_gather(c1_v, [cv])
        w_v[pl.ds(i, 16)] = 1.0 / jnp.maximum(g0 + g1, 1.0)

    pltpu.sync_copy(w_v, w_hbm.at[pl.ds(wid * EPT, EPT)])


@functools.partial(
    pl.kernel,
    out_type=jax.ShapeDtypeStruct((NC, N, D), jnp.float32),
    mesh=_mesh,
    scratch_types=[
        [pltpu.VMEM((KCH,), jnp.int32) for _ in range(3)],
        [pltpu.VMEM((KCH,), jnp.int32) for _ in range(3)],
        [pltpu.VMEM((KCH,), jnp.int32) for _ in range(3)],
        [pltpu.VMEM((KCH,), jnp.float32) for _ in range(3)],
        [pltpu.VMEM((KCH, D), jnp.float32) for _ in range(3)],
        pltpu.VMEM_SHARED((N, D), jnp.float32),
        [pltpu.SemaphoreType.DMA for _ in range(3)],
        [pltpu.SemaphoreType.DMA for _ in range(3)],
        [pltpu.SemaphoreType.DMA for _ in range(3)],
    ],
    compiler_params=_sc_params,
)
def _k3_aggregate(hrw_hbm, fidx_hbm, dst_hbm, w_hbm, bsum_hbm,
                  gidx, sidx, sidx2, wv, rows, bkt_sh, rsem, isem, ssem):
    c = lax.axis_index("c")
    s = lax.axis_index("s")
    wid = c * NS + s
    base = wid * EPTP

    @pl.loop(0, KCH)
    def _(i):
        for j in range(D // 16):
            rows[0][i, pl.ds(j * 16, 16)] = _zero16()

    def _zero_rows(start, cnt):
        off = 0
        while cnt - off >= KCH:
            pltpu.sync_copy(rows[0], bkt_sh.at[pl.ds(start + off, KCH)])
            off += KCH
        if cnt > off:
            pltpu.sync_copy(rows[0].at[pl.ds(0, cnt - off)],
                            bkt_sh.at[pl.ds(start + off, cnt - off)])

    _zero_rows(s * NPT, NPT)

    @pl.when(s == NS - 1)
    def _():
        _zero_rows(NS * NPT, N - NS * NPT)

    plsc.subcore_barrier()

    def _idx_start(k, x):
        off = base + k * KCH
        pltpu.async_copy(fidx_hbm.at[pl.ds(off, KCH)], gidx[x], isem[x])
        pltpu.async_copy(dst_hbm.at[pl.ds(off, KCH)], sidx[x], isem[x])
        pltpu.async_copy(w_hbm.at[pl.ds(off, KCH)], wv[x], isem[x])

    def _idx_wait(k, x):
        off = base + k * KCH
        pltpu.make_async_copy(fidx_hbm.at[pl.ds(off, KCH)], gidx[x],
                              isem[x]).wait()
        pltpu.make_async_copy(dst_hbm.at[pl.ds(off, KCH)], sidx[x],
                              isem[x]).wait()
        pltpu.make_async_copy(w_hbm.at[pl.ds(off, KCH)], wv[x],
                              isem[x]).wait()

    def _gather_start(x):
        pltpu.async_copy(hrw_hbm.at[gidx[x]], rows[x], rsem[x])

    def _scatter_wait(y):
        pltpu.make_async_copy(rows[y], bkt_sh.at[sidx2[y]], ssem[y]).wait()

    def _visit(k, x, y, prefetch=True, swait=True):
        pltpu.make_async_copy(hrw_hbm.at[gidx[x]], rows[x], rsem[x]).wait()
        buf = rows[x]

        @pl.loop(0, KCH)
        def _(i):
            ww = plsc.load_gather(wv[x], [jnp.full((16,), i, jnp.int32)])
            for jj in range(D // 16):
                sl = pl.ds(jj * 16, 16)
                buf[i, sl] = buf[i, sl] * ww

        @pl.loop(0, KCH, step=16)
        def _(i):
            sidx2[x][pl.ds(i, 16)] = sidx[x][pl.ds(i, 16)]

        pltpu.async_copy(buf, bkt_sh.at[sidx2[x]], ssem[x], add=True)

        if prefetch:
            @pl.when(k + 3 < NCH)
            def _():
                _idx_start(k + 3, x)

            _idx_wait(k + 2, y)
            if swait:
                _scatter_wait(y)
            _gather_start(y)

    _idx_start(0, 0)
    _idx_start(1, 1)
    _idx_start(2, 2)
    _idx_wait(0, 0)
    _gather_start(0)
    _idx_wait(1, 1)
    _gather_start(1)

    _NTRIPS = (NCH // 3) * 3

    _visit(0, 0, 2, swait=False)
    _visit(1, 1, 0)
    _visit(2, 2, 1)

    @pl.loop(3, _NTRIPS, step=3)
    def _(k):
        _visit(k, 0, 2)
        _visit(k + 1, 1, 0)
        _visit(k + 2, 2, 1)

    _visit(_NTRIPS, 0, 2, prefetch=False)
    _visit(_NTRIPS + 1, 1, 0, prefetch=False)

    _scatter_wait(0)
    _scatter_wait(1)
    _scatter_wait(2)

    plsc.subcore_barrier()
    pltpu.sync_copy(bkt_sh.at[pl.ds(s * NPT, NPT)],
                    bsum_hbm.at[c, pl.ds(s * NPT, NPT)])

    @pl.when(s == NS - 1)
    def _():
        pltpu.sync_copy(bkt_sh.at[pl.ds(NS * NPT, N - NS * NPT)],
                        bsum_hbm.at[c, pl.ds(NS * NPT, N - NS * NPT)])


_NB = 10
_BN = N // _NB


def _tca_body(x_ref, w_ref, o_ref):
    o_ref[0] = jnp.dot(x_ref[...], w_ref[0],
                       preferred_element_type=jnp.float32)


def _tca(h, w):
    return pl.pallas_call(
        _tca_body,
        grid=(R, _NB),
        in_specs=[
            pl.BlockSpec((_BN, D), lambda r, i: (i, 0)),
            pl.BlockSpec((1, D, D), lambda r, i: (r, 0, 0)),
        ],
        out_specs=pl.BlockSpec((1, _BN, D), lambda r, i: (r, i, 0)),
        out_shape=jax.ShapeDtypeStruct((R, N, D), jnp.float32),
    )(h, w)


def _tcb_body(bsum_ref, h_ref, root_ref, b_ref, wn_ref, h_out, hrw_out):
    hn = bsum_ref[0] + bsum_ref[1]
    hn = hn + jnp.dot(h_ref[...], root_ref[...],
                      preferred_element_type=jnp.float32)
    hn = jnp.maximum(hn + b_ref[...], 0.0)
    h_out[...] = hn
    if hrw_out is not None:
        for r in range(R):
            hrw_out[r] = jnp.dot(hn, wn_ref[r],
                                 preferred_element_type=jnp.float32)


def _tcb(bsum, h, root, b, wnext):
    b2 = b.reshape(1, D)
    in_specs = [
        pl.BlockSpec((NC, _BN, D), lambda i: (0, i, 0)),
        pl.BlockSpec((_BN, D), lambda i: (i, 0)),
        pl.BlockSpec((D, D), lambda i: (0, 0)),
        pl.BlockSpec((1, D), lambda i: (0, 0)),
    ]
    if wnext is None:
        body = lambda bs, hh, rt, bb, ho: _tcb_body(bs, hh, rt, bb, None,
                                                    ho, None)
        return pl.pallas_call(
            body,
            grid=(_NB,),
            in_specs=in_specs,
            out_specs=pl.BlockSpec((_BN, D), lambda i: (i, 0)),
            out_shape=jax.ShapeDtypeStruct((N, D), jnp.float32),
        )(bsum, h, root, b2)
    return pl.pallas_call(
        _tcb_body,
        grid=(_NB,),
        in_specs=in_specs + [pl.BlockSpec((R, D, D), lambda i: (0, 0, 0))],
        out_specs=[
            pl.BlockSpec((_BN, D), lambda i: (i, 0)),
            pl.BlockSpec((R, _BN, D), lambda i: (0, i, 0)),
        ],
        out_shape=[
            jax.ShapeDtypeStruct((N, D), jnp.float32),
            jax.ShapeDtypeStruct((R, N, D), jnp.float32),
        ],
    )(bsum, h, root, b2, wnext)


def kernel(x, edge_index, edge_type, weights, roots, biases):
    src = edge_index[0].astype(jnp.int32)
    dst = edge_index[1].astype(jnp.int32)
    et = edge_type.astype(jnp.int32)
    comb = dst * R + et
    fidx = et * N + src

    cnt = _k1_counts(comb)
    w = _k2_weights(cnt, comb)

    npad = EPTP - EPT
    pad_g = (jnp.arange(npad, dtype=jnp.int32)[None, :] * 131
             + jnp.arange(NW, dtype=jnp.int32)[:, None] * 17) % (R * N)
    pad_s = (jnp.arange(npad, dtype=jnp.int32)[None, :] * 41
             + jnp.arange(NW, dtype=jnp.int32)[:, None] * 13) % N
    pad_w = jnp.zeros((NW, npad), jnp.float32)
    fidx_p = jnp.concatenate([fidx.reshape(NW, EPT), pad_g], axis=1).reshape(EP)
    dst_p = jnp.concatenate([dst.reshape(NW, EPT), pad_s], axis=1).reshape(EP)
    w_p = jnp.concatenate([w.reshape(NW, EPT), pad_w], axis=1).reshape(EP)

    h = x
    hrw = _tca(h, weights[0]).reshape(R * N, D)
    for l in range(L):
        bsum = _k3_aggregate(hrw, fidx_p, dst_p, w_p)
        if l < L - 1:
            h, hrw3 = _tcb(bsum, h, roots[l], biases[l], weights[l + 1])
            hrw = hrw3.reshape(R * N, D)
        else:
            h = _tcb(bsum, h, roots[l], biases[l], None)
    return h
